# baseline (speedup 1.0000x reference)
import functools

import jax
import jax.numpy as jnp
from jax.experimental import pallas as pl
from jax.experimental.pallas import tpu as pltpu

BATCH = 1024
D = 128
NUM_CLASS = 100000
BM = 32
NUM_BANDS = BATCH // BM
CK = 2048
FULL_CHUNKS = NUM_CLASS // CK
TAIL = NUM_CLASS - FULL_CHUNKS * CK
NBUF = 2


def _band_copy(o_ref, band_ref, sem_ref, step):
    slot = jax.lax.rem(step, NBUF)
    return pltpu.make_async_copy(
        band_ref.at[slot],
        o_ref.at[pl.ds(step * BM, BM), :],
        sem_ref.at[slot],
    )


def _mm_kernel(x_ref, wt_ref, b_ref, o_ref, band_ref, sem_ref):
    m = pl.program_id(0)
    slot = jax.lax.rem(m, NBUF)

    @pl.when(m >= NBUF)
    def _():
        _band_copy(o_ref, band_ref, sem_ref, m - NBUF).wait()

    xb = x_ref[...]
    for k in range(FULL_CHUNKS + 1):
        lo = k * CK
        width = CK if k < FULL_CHUNKS else TAIL
        acc = jax.lax.dot_general(
            xb, wt_ref[:, pl.ds(lo, width)],
            dimension_numbers=(((1,), (0,)), ((), ())),
            preferred_element_type=jnp.float32,
        )
        band_ref[slot, :, pl.ds(lo, width)] = acc + b_ref[:, pl.ds(lo, width)]

    _band_copy(o_ref, band_ref, sem_ref, m).start()

    @pl.when(m == NUM_BANDS - 1)
    def _():
        for j in range(NBUF - 1, -1, -1):
            _band_copy(o_ref, band_ref, sem_ref, NUM_BANDS - 1 - j).wait()


@functools.partial(jax.jit, static_argnames=())
def _lsh_eval_forward(x, W, b):
    x16 = x.astype(jnp.bfloat16)
    w16t = W.astype(jnp.bfloat16).T
    b_row = jnp.reshape(b, (1, NUM_CLASS))
    return pl.pallas_call(
        _mm_kernel,
        grid=(NUM_BANDS,),
        in_specs=[
            pl.BlockSpec((BM, D), lambda m: (m, 0)),
            pl.BlockSpec(memory_space=pltpu.VMEM),
            pl.BlockSpec(memory_space=pltpu.VMEM),
        ],
        out_specs=pl.BlockSpec(memory_space=pltpu.HBM),
        out_shape=jax.ShapeDtypeStruct((BATCH, NUM_CLASS), jnp.float32),
        scratch_shapes=[
            pltpu.VMEM((NBUF, BM, NUM_CLASS), jnp.float32),
            pltpu.SemaphoreType.DMA((NBUF,)),
        ],
        compiler_params=pltpu.CompilerParams(
            dimension_semantics=(pltpu.ARBITRARY,),
            vmem_limit_bytes=63 * 1024 * 1024,
        ),
    )(x16, w16t, b_row)


def kernel(x, y, triplet_flag, debug, W, b):
    del y, triplet_flag, debug
    return _lsh_eval_forward(x, W, b)


# speedup vs baseline: 1.0087x; 1.0087x over previous
import functools

import jax
import jax.numpy as jnp
from jax.experimental import pallas as pl
from jax.experimental.pallas import tpu as pltpu

BATCH = 1024
D = 128
NUM_CLASS = 100000
BM = 32
NUM_BANDS = BATCH // BM
CK = 4096
FULL_CHUNKS = NUM_CLASS // CK
TAIL = NUM_CLASS - FULL_CHUNKS * CK
NBUF = 3


def _band_copy(o_ref, band_ref, sem_ref, step):
    slot = jax.lax.rem(step, NBUF)
    return pltpu.make_async_copy(
        band_ref.at[slot],
        o_ref.at[pl.ds(step * BM, BM), :],
        sem_ref.at[slot],
    )


def _mm_kernel(x_ref, wt_ref, b_ref, o_ref, band_ref, sem_ref):
    m = pl.program_id(0)
    slot = jax.lax.rem(m, NBUF)

    @pl.when(m >= NBUF)
    def _():
        _band_copy(o_ref, band_ref, sem_ref, m - NBUF).wait()

    xb = x_ref[...]
    for k in range(FULL_CHUNKS + 1):
        lo = k * CK
        width = CK if k < FULL_CHUNKS else TAIL
        acc = jax.lax.dot_general(
            xb, wt_ref[:, pl.ds(lo, width)],
            dimension_numbers=(((1,), (0,)), ((), ())),
            preferred_element_type=jnp.float32,
        )
        band_ref[slot, :, pl.ds(lo, width)] = acc + b_ref[:, pl.ds(lo, width)]

    _band_copy(o_ref, band_ref, sem_ref, m).start()

    @pl.when(m == NUM_BANDS - 1)
    def _():
        for j in range(NBUF - 1, -1, -1):
            _band_copy(o_ref, band_ref, sem_ref, NUM_BANDS - 1 - j).wait()


@functools.partial(jax.jit, static_argnames=())
def _lsh_eval_forward(x, W, b):
    x16 = x.astype(jnp.bfloat16)
    w16t = W.astype(jnp.bfloat16).T
    b_row = jnp.reshape(b, (1, NUM_CLASS))
    return pl.pallas_call(
        _mm_kernel,
        grid=(NUM_BANDS,),
        in_specs=[
            pl.BlockSpec((BM, D), lambda m: (m, 0)),
            pl.BlockSpec(memory_space=pltpu.VMEM),
            pl.BlockSpec(memory_space=pltpu.VMEM),
        ],
        out_specs=pl.BlockSpec(memory_space=pltpu.HBM),
        out_shape=jax.ShapeDtypeStruct((BATCH, NUM_CLASS), jnp.float32),
        scratch_shapes=[
            pltpu.VMEM((NBUF, BM, NUM_CLASS), jnp.float32),
            pltpu.SemaphoreType.DMA((NBUF,)),
        ],
        compiler_params=pltpu.CompilerParams(
            dimension_semantics=(pltpu.ARBITRARY,),
            vmem_limit_bytes=63 * 1024 * 1024,
        ),
    )(x16, w16t, b_row)


def kernel(x, y, triplet_flag, debug, W, b):
    del y, triplet_flag, debug
    return _lsh_eval_forward(x, W, b)
